# baseline (speedup 1.0000x reference)
import functools

import jax
import jax.numpy as jnp
from jax import lax
from jax.experimental import pallas as pl
from jax.experimental.pallas import tpu as pltpu
from jax.experimental.pallas import tpu_sc as plsc

NC = 2
NS = 16
SUB = 128


def _segment_sum_sc(dest2d, msg3, zeros):
    n = zeros.shape[0]
    d_msg = 16
    num_blocks = msg3.shape[0] // (2 * 8 * SUB)
    bpu = 2
    blk = 8 * SUB
    num_units = num_blocks // bpu
    rows_per_sub = ((n + NS * 8 - 1) // (NS * 8)) * 8

    mesh = plsc.VectorSubcoreMesh(core_axis_name="c", subcore_axis_name="s")

    vm = pltpu.VMEM
    @functools.partial(
        pl.kernel,
        mesh=mesh,
        compiler_params=pltpu.CompilerParams(use_tc_tiling_on_sc=False,
                                             needs_layout_passes=False),
        out_type=jax.ShapeDtypeStruct((NC, n, 128), jnp.float32),
        scratch_types=[
            vm((bpu * blk * 2,), jnp.float32), vm((bpu * blk * 2,), jnp.float32),
            vm((bpu * SUB, d_msg), jnp.float32), vm((bpu * SUB, d_msg), jnp.float32),
            vm((bpu, SUB), jnp.int32), vm((bpu, SUB), jnp.int32),
            vm((bpu, SUB), jnp.int32), vm((bpu, SUB), jnp.int32),
            pltpu.SemaphoreType.DMA, pltpu.SemaphoreType.DMA,
            pltpu.SemaphoreType.DMA, pltpu.SemaphoreType.DMA,
            pltpu.VMEM_SHARED((n, d_msg), jnp.float32),
        ],
    )
    def seg_sum(dest_hbm, msg_hbm, zeros_hbm, out_hbm,
                stg0, stg1, msgb0, msgb1, idx0, idx1, sidx0, sidx1,
                isem0, isem1, ssem0, ssem1, acc):
        stg = (stg0, stg1)
        msgb = (msgb0, msgb1)
        idx = (idx0, idx1)
        sidx = (sidx0, sidx1)
        isem = (isem0, isem1)
        ssem = (ssem0, ssem1)

        c = lax.axis_index("c")
        s = lax.axis_index("s")
        wid = c * NS + s

        row0 = pl.multiple_of(
            jnp.minimum(s * rows_per_sub, n - rows_per_sub), 8)
        pltpu.sync_copy(zeros_hbm.at[pl.ds(row0, rows_per_sub)],
                        acc.at[pl.ds(row0, rows_per_sub)])
        plsc.subcore_barrier()

        base = num_units // (NC * NS)
        rem = num_units % (NC * NS)
        extra = jnp.where(wid < rem, 1, 0)
        start = wid * base + jnp.minimum(wid, rem)
        count = base + extra

        f_iota = lax.iota(jnp.int32, 16)
        base_vec = (f_iota % 8) * SUB + (f_iota // 8) * (bpu * blk)

        def in_copies(u, b):
            yield dest_hbm.at[pl.ds(u * bpu, bpu)], idx[b], isem[b]
            yield (msg_hbm.at[pl.ds(u * bpu * blk, bpu * blk)],
                   stg[b].at[pl.ds(0, bpu * blk)], isem[b])
            yield (msg_hbm.at[pl.ds((num_blocks + u * bpu) * blk, bpu * blk)],
                   stg[b].at[pl.ds(bpu * blk, bpu * blk)], isem[b])

        def sc_copies(b):
            for k in range(bpu):
                yield (msgb[b].at[pl.ds(k * SUB, SUB)], acc.at[sidx[b].at[k]],
                       ssem[b])

        def pbody(p, _):
            for b in range(2):
                i = 2 * p + b
                u = start + i

                @pl.when((i >= 2) & (i < count + 2))
                def _():
                    for a_, d_, m_ in sc_copies(b):
                        pltpu.make_async_copy(a_, d_, m_).wait()

                @pl.when(i < count)
                def _():
                    for a_, d_, m_ in in_copies(u, b):
                        pltpu.make_async_copy(a_, d_, m_).wait()
                    for k in range(bpu):
                        for j in range(SUB // 16):
                            sidx[b][k, pl.ds(j * 16, 16)] = (
                                idx[b][k, pl.ds(j * 16, 16)])
                    @plsc.parallel_loop(0, SUB, 8, unroll=2, carry=base_vec)
                    def _(e0, idxc):
                        for e2 in range(8):
                            for k in range(bpu):
                                msgb[b][k * SUB + e0 + e2] = plsc.load_gather(
                                    stg[b], [idxc + (k * blk + e2)])
                        return idxc + 8
                    for a_, d_, m_ in sc_copies(b):
                        pltpu.async_copy(a_, d_, m_, add=True)

                @pl.when(i < count - 2)
                def _():
                    for a_, d_, m_ in in_copies(u + 2, b):
                        pltpu.async_copy(a_, d_, m_)
            return 0

        for a_, d_, m_ in in_copies(start, 0):
            pltpu.async_copy(a_, d_, m_)
        for a_, d_, m_ in in_copies(start + 1, 1):
            pltpu.async_copy(a_, d_, m_)
        lax.fori_loop(0, (count + 3) // 2, pbody, 0)

        plsc.subcore_barrier()
        pltpu.sync_copy(acc.at[pl.ds(row0, rows_per_sub)],
                        out_hbm.at[c].at[pl.ds(row0, rows_per_sub),
                                         pl.ds(0, d_msg)])

    return seg_sum(dest2d, msg3, zeros)


def _mlp_body(x_ref, p_ref, w1x_ref, w1m_ref, b1_ref, w2_ref,
              b2_ref, w3_ref, g_ref, be_ref, o_ref):
    xb = x_ref[...]
    m = p_ref[0, :, :16] + p_ref[1, :, :16]
    h = (jnp.dot(xb, w1x_ref[...], preferred_element_type=jnp.float32)
         + jnp.dot(m, w1m_ref[...], preferred_element_type=jnp.float32)
         + b1_ref[...])
    h = jnp.where(h >= 0, h, 0.2 * h)
    h = jnp.dot(h, w2_ref[...], preferred_element_type=jnp.float32) + b2_ref[...]
    h = jnp.where(h >= 0, h, 0.2 * h)
    h = jnp.dot(h, w3_ref[...], preferred_element_type=jnp.float32)
    mu = jnp.mean(h, axis=-1, keepdims=True)
    var = jnp.mean((h - mu) ** 2, axis=-1, keepdims=True)
    o_ref[...] = (h - mu) * lax.rsqrt(var + 1e-5) * g_ref[...] + be_ref[...]


def _mlp_tc(x, parts, W1x, W1m, b1, W2, b2, W3, gamma, beta, block_n):
    n, d_in = x.shape
    d_msg = W1m.shape[0]
    d_out = W1x.shape[1]
    grid = (n // block_n,)
    return pl.pallas_call(
        _mlp_body,
        grid=grid,
        in_specs=[
            pl.BlockSpec((block_n, d_in), lambda i: (i, 0)),
            pl.BlockSpec((2, block_n, 128), lambda i: (0, i, 0)),
            pl.BlockSpec((d_in, d_out), lambda i: (0, 0)),
            pl.BlockSpec((d_msg, d_out), lambda i: (0, 0)),
            pl.BlockSpec((1, d_out), lambda i: (0, 0)),
            pl.BlockSpec((d_out, d_out), lambda i: (0, 0)),
            pl.BlockSpec((1, d_out), lambda i: (0, 0)),
            pl.BlockSpec((d_out, d_out), lambda i: (0, 0)),
            pl.BlockSpec((1, d_out), lambda i: (0, 0)),
            pl.BlockSpec((1, d_out), lambda i: (0, 0)),
        ],
        out_specs=pl.BlockSpec((block_n, d_out), lambda i: (i, 0)),
        out_shape=jax.ShapeDtypeStruct((n, d_out), jnp.float32),
        compiler_params=pltpu.CompilerParams(
            dimension_semantics=("arbitrary",),
        ),
    )(x, parts, W1x, W1m, b1, W2, b2, W3, gamma, beta)


def kernel(x, edge_index, message, W1, b1, W2, b2, W3, gamma, beta):
    n, d_in = x.shape
    e, d_msg = message.shape

    dest2d = edge_index[1].reshape(e // SUB, SUB)
    msg3 = (message.T.reshape(2, 8, e // SUB, SUB)
            .transpose(0, 2, 1, 3).reshape(-1))
    zeros = jnp.zeros((n, d_msg), dtype=jnp.float32)
    parts = _segment_sum_sc(dest2d, msg3, zeros)

    W1x = W1[:d_in]
    W1m = W1[d_in:]
    out = _mlp_tc(x, parts, W1x, W1m, b1.reshape(1, -1),
                  W2, b2.reshape(1, -1), W3, gamma.reshape(1, -1),
                  beta.reshape(1, -1), block_n=2000)
    return out


# speedup vs baseline: 1.0098x; 1.0098x over previous
import functools

import jax
import jax.numpy as jnp
from jax import lax
from jax.experimental import pallas as pl
from jax.experimental.pallas import tpu as pltpu
from jax.experimental.pallas import tpu_sc as plsc

NC = 2
NS = 16
SUB = 128


def _segment_sum_sc(dest2d, msg3, zeros):
    n = zeros.shape[0]
    d_msg = 16
    num_blocks = msg3.shape[0] // (2 * 8 * SUB)
    bpu = 2
    blk = 8 * SUB
    num_units = num_blocks // bpu
    rows_per_sub = ((n + NS * 8 - 1) // (NS * 8)) * 8

    mesh = plsc.VectorSubcoreMesh(core_axis_name="c", subcore_axis_name="s")

    vm = pltpu.VMEM
    @functools.partial(
        pl.kernel,
        mesh=mesh,
        compiler_params=pltpu.CompilerParams(use_tc_tiling_on_sc=False,
                                             needs_layout_passes=False),
        out_type=jax.ShapeDtypeStruct((n, 128), jnp.float32),
        scratch_types=[
            vm((bpu * blk * 2,), jnp.float32), vm((bpu * blk * 2,), jnp.float32),
            vm((bpu * SUB, d_msg), jnp.float32), vm((bpu * SUB, d_msg), jnp.float32),
            vm((bpu, SUB), jnp.int32), vm((bpu, SUB), jnp.int32),
            vm((bpu, SUB), jnp.int32), vm((bpu, SUB), jnp.int32),
            pltpu.SemaphoreType.DMA, pltpu.SemaphoreType.DMA,
            pltpu.SemaphoreType.DMA, pltpu.SemaphoreType.DMA,
            pltpu.VMEM_SHARED((n, d_msg), jnp.float32),
        ],
    )
    def seg_sum(dest_hbm, msg_hbm, zeros_hbm, out_hbm,
                stg0, stg1, msgb0, msgb1, idx0, idx1, sidx0, sidx1,
                isem0, isem1, ssem0, ssem1, acc):
        stg = (stg0, stg1)
        msgb = (msgb0, msgb1)
        idx = (idx0, idx1)
        sidx = (sidx0, sidx1)
        isem = (isem0, isem1)
        ssem = (ssem0, ssem1)

        c = lax.axis_index("c")
        s = lax.axis_index("s")
        wid = c * NS + s

        row0 = pl.multiple_of(
            jnp.minimum(s * rows_per_sub, n - rows_per_sub), 8)
        pltpu.sync_copy(zeros_hbm.at[pl.ds(row0, rows_per_sub)],
                        acc.at[pl.ds(row0, rows_per_sub)])
        plsc.subcore_barrier()

        base = num_units // (NC * NS)
        rem = num_units % (NC * NS)
        extra = jnp.where(wid < rem, 1, 0)
        start = wid * base + jnp.minimum(wid, rem)
        count = base + extra

        f_iota = lax.iota(jnp.int32, 16)
        base_vec = (f_iota % 8) * SUB + (f_iota // 8) * (bpu * blk)

        def in_copies(u, b):
            yield dest_hbm.at[pl.ds(u * bpu, bpu)], idx[b], isem[b]
            yield (msg_hbm.at[pl.ds(u * bpu * blk, bpu * blk)],
                   stg[b].at[pl.ds(0, bpu * blk)], isem[b])
            yield (msg_hbm.at[pl.ds((num_blocks + u * bpu) * blk, bpu * blk)],
                   stg[b].at[pl.ds(bpu * blk, bpu * blk)], isem[b])

        def sc_copies(b):
            for k in range(bpu):
                yield (msgb[b].at[pl.ds(k * SUB, SUB)], acc.at[sidx[b].at[k]],
                       ssem[b])

        def pbody(p, _):
            for b in range(2):
                i = 2 * p + b
                u = start + i

                @pl.when((i >= 2) & (i < count + 2))
                def _():
                    for a_, d_, m_ in sc_copies(b):
                        pltpu.make_async_copy(a_, d_, m_).wait()

                @pl.when(i < count)
                def _():
                    for a_, d_, m_ in in_copies(u, b):
                        pltpu.make_async_copy(a_, d_, m_).wait()
                    for k in range(bpu):
                        for j in range(SUB // 16):
                            sidx[b][k, pl.ds(j * 16, 16)] = (
                                idx[b][k, pl.ds(j * 16, 16)])
                    @plsc.parallel_loop(0, SUB, 8, carry=base_vec)
                    def _(e0, idxc):
                        for e2 in range(8):
                            for k in range(bpu):
                                msgb[b][k * SUB + e0 + e2] = plsc.load_gather(
                                    stg[b], [idxc + (k * blk + e2)])
                        return idxc + 8
                    for a_, d_, m_ in sc_copies(b):
                        pltpu.async_copy(a_, d_, m_, add=True)

                @pl.when(i < count - 2)
                def _():
                    for a_, d_, m_ in in_copies(u + 2, b):
                        pltpu.async_copy(a_, d_, m_)
            return 0

        for a_, d_, m_ in in_copies(start, 0):
            pltpu.async_copy(a_, d_, m_)
        for a_, d_, m_ in in_copies(start + 1, 1):
            pltpu.async_copy(a_, d_, m_)
        lax.fori_loop(0, (count + 3) // 2, pbody, 0)

        plsc.subcore_barrier()
        pltpu.sync_copy(acc.at[pl.ds(row0, rows_per_sub)],
                        out_hbm.at[pl.ds(row0, rows_per_sub),
                                   pl.ds(pl.multiple_of(c * d_msg, 16),
                                         d_msg)])

    return seg_sum(dest2d, msg3, zeros)


def _mlp_body(x_ref, p_ref, w1x_ref, w1m_ref, b1_ref, w2_ref,
              b2_ref, w3_ref, g_ref, be_ref, o_ref):
    xb = x_ref[...]
    m = p_ref[:, :16] + p_ref[:, 16:32]
    h = (jnp.dot(xb, w1x_ref[...], preferred_element_type=jnp.float32)
         + jnp.dot(m, w1m_ref[...], preferred_element_type=jnp.float32)
         + b1_ref[...])
    h = jnp.where(h >= 0, h, 0.2 * h)
    h = jnp.dot(h, w2_ref[...], preferred_element_type=jnp.float32) + b2_ref[...]
    h = jnp.where(h >= 0, h, 0.2 * h)
    h = jnp.dot(h, w3_ref[...], preferred_element_type=jnp.float32)
    mu = jnp.mean(h, axis=-1, keepdims=True)
    var = jnp.mean((h - mu) ** 2, axis=-1, keepdims=True)
    o_ref[...] = (h - mu) * lax.rsqrt(var + 1e-5) * g_ref[...] + be_ref[...]


def _mlp_tc(x, parts, W1x, W1m, b1, W2, b2, W3, gamma, beta, block_n):
    n, d_in = x.shape
    d_msg = W1m.shape[0]
    d_out = W1x.shape[1]
    grid = (n // block_n,)
    return pl.pallas_call(
        _mlp_body,
        grid=grid,
        in_specs=[
            pl.BlockSpec((block_n, d_in), lambda i: (i, 0)),
            pl.BlockSpec((block_n, 128), lambda i: (i, 0)),
            pl.BlockSpec((d_in, d_out), lambda i: (0, 0)),
            pl.BlockSpec((d_msg, d_out), lambda i: (0, 0)),
            pl.BlockSpec((1, d_out), lambda i: (0, 0)),
            pl.BlockSpec((d_out, d_out), lambda i: (0, 0)),
            pl.BlockSpec((1, d_out), lambda i: (0, 0)),
            pl.BlockSpec((d_out, d_out), lambda i: (0, 0)),
            pl.BlockSpec((1, d_out), lambda i: (0, 0)),
            pl.BlockSpec((1, d_out), lambda i: (0, 0)),
        ],
        out_specs=pl.BlockSpec((block_n, d_out), lambda i: (i, 0)),
        out_shape=jax.ShapeDtypeStruct((n, d_out), jnp.float32),
        compiler_params=pltpu.CompilerParams(
            dimension_semantics=("arbitrary",),
        ),
    )(x, parts, W1x, W1m, b1, W2, b2, W3, gamma, beta)


def kernel(x, edge_index, message, W1, b1, W2, b2, W3, gamma, beta):
    n, d_in = x.shape
    e, d_msg = message.shape

    dest2d = edge_index[1].reshape(e // SUB, SUB)
    msg3 = (message.T.reshape(2, 8, e // SUB, SUB)
            .transpose(0, 2, 1, 3).reshape(-1))
    zeros = jnp.zeros((n, d_msg), dtype=jnp.float32)
    parts = _segment_sum_sc(dest2d, msg3, zeros)

    W1x = W1[:d_in]
    W1m = W1[d_in:]
    out = _mlp_tc(x, parts, W1x, W1m, b1.reshape(1, -1),
                  W2, b2.reshape(1, -1), W3, gamma.reshape(1, -1),
                  beta.reshape(1, -1), block_n=2000)
    return out


# speedup vs baseline: 2.0995x; 2.0791x over previous
import functools

import jax
import jax.numpy as jnp
from jax import lax
from jax.experimental import pallas as pl
from jax.experimental.pallas import tpu as pltpu
from jax.experimental.pallas import tpu_sc as plsc

NC = 2
NS = 16
SUB = 128


def _segment_sum_sc(dest2d, msg3, zeros):
    n = zeros.shape[0]
    d_msg = 16
    num_blocks = msg3.shape[0] // (2 * 8)
    bpu = 2
    blk = 8 * SUB
    num_units = num_blocks // bpu
    rows_per_sub = ((n + NS * 8 - 1) // (NS * 8)) * 8

    mesh = plsc.VectorSubcoreMesh(core_axis_name="c", subcore_axis_name="s")

    vm = pltpu.VMEM
    @functools.partial(
        pl.kernel,
        mesh=mesh,
        compiler_params=pltpu.CompilerParams(use_tc_tiling_on_sc=False,
                                             needs_layout_passes=False),
        out_type=jax.ShapeDtypeStruct((n, 128), jnp.float32),
        scratch_types=[
            vm((bpu * 16, 129), jnp.float32), vm((bpu * 16, 129), jnp.float32),
            vm((bpu * SUB, d_msg), jnp.float32), vm((bpu * SUB, d_msg), jnp.float32),
            vm((bpu, SUB), jnp.int32), vm((bpu, SUB), jnp.int32),
            vm((bpu, SUB), jnp.int32), vm((bpu, SUB), jnp.int32),
            pltpu.SemaphoreType.DMA, pltpu.SemaphoreType.DMA,
            pltpu.SemaphoreType.DMA, pltpu.SemaphoreType.DMA,
            pltpu.VMEM_SHARED((n, d_msg), jnp.float32),
        ],
    )
    def seg_sum(dest_hbm, msg_hbm, zeros_hbm, out_hbm,
                stg0, stg1, msgb0, msgb1, idx0, idx1, sidx0, sidx1,
                isem0, isem1, ssem0, ssem1, acc):
        stg = (stg0, stg1)
        msgb = (msgb0, msgb1)
        idx = (idx0, idx1)
        sidx = (sidx0, sidx1)
        isem = (isem0, isem1)
        ssem = (ssem0, ssem1)

        c = lax.axis_index("c")
        s = lax.axis_index("s")
        wid = c * NS + s

        row0 = pl.multiple_of(
            jnp.minimum(s * rows_per_sub, n - rows_per_sub), 8)
        pltpu.sync_copy(zeros_hbm.at[pl.ds(row0, rows_per_sub)],
                        acc.at[pl.ds(row0, rows_per_sub)])
        plsc.subcore_barrier()

        base = num_units // (NC * NS)
        rem = num_units % (NC * NS)
        extra = jnp.where(wid < rem, 1, 0)
        start = wid * base + jnp.minimum(wid, rem)
        count = base + extra

        f_iota = lax.iota(jnp.int32, 16)
        row_const = [k * 16 + f_iota for k in range(bpu)]
        zero_vec = f_iota * 0

        def in_copies(u, b):
            yield dest_hbm.at[pl.ds(u * bpu, bpu)], idx[b], isem[b]
            for k in range(bpu):
                for h in range(2):
                    src_row = (h * num_blocks + u * bpu + k) * 8
                    yield (msg_hbm.at[pl.ds(src_row, 8)],
                           stg[b].at[pl.ds(k * 16 + h * 8, 8), pl.ds(0, SUB)],
                           isem[b])

        def sc_copies(b):
            for k in range(bpu):
                yield (msgb[b].at[pl.ds(k * SUB, SUB)], acc.at[sidx[b].at[k]],
                       ssem[b])

        def pbody(p, _):
            for b in range(2):
                i = 2 * p + b
                u = start + i

                @pl.when((i >= 2) & (i < count + 2))
                def _():
                    for a_, d_, m_ in sc_copies(b):
                        pltpu.make_async_copy(a_, d_, m_).wait()

                @pl.when(i < count)
                def _():
                    for a_, d_, m_ in in_copies(u, b):
                        pltpu.make_async_copy(a_, d_, m_).wait()
                    for k in range(bpu):
                        for j in range(SUB // 16):
                            sidx[b][k, pl.ds(j * 16, 16)] = (
                                idx[b][k, pl.ds(j * 16, 16)])
                    @plsc.parallel_loop(0, SUB, 8, carry=zero_vec)
                    def _(e0, ecol):
                        for e2 in range(8):
                            for k in range(bpu):
                                msgb[b][k * SUB + e0 + e2] = plsc.load_gather(
                                    stg[b], [row_const[k], ecol + e2])
                        return ecol + 8
                    for a_, d_, m_ in sc_copies(b):
                        pltpu.async_copy(a_, d_, m_, add=True)

                @pl.when(i < count - 2)
                def _():
                    for a_, d_, m_ in in_copies(u + 2, b):
                        pltpu.async_copy(a_, d_, m_)
            return 0

        for a_, d_, m_ in in_copies(start, 0):
            pltpu.async_copy(a_, d_, m_)
        for a_, d_, m_ in in_copies(start + 1, 1):
            pltpu.async_copy(a_, d_, m_)
        lax.fori_loop(0, (count + 3) // 2, pbody, 0)

        plsc.subcore_barrier()
        pltpu.sync_copy(acc.at[pl.ds(row0, rows_per_sub)],
                        out_hbm.at[pl.ds(row0, rows_per_sub),
                                   pl.ds(pl.multiple_of(c * d_msg, 16),
                                         d_msg)])

    return seg_sum(dest2d, msg3, zeros)


def _mlp_body(x_ref, p_ref, w1x_ref, w1m_ref, b1_ref, w2_ref,
              b2_ref, w3_ref, g_ref, be_ref, o_ref):
    xb = x_ref[...]
    m = p_ref[:, :16] + p_ref[:, 16:32]
    h = (jnp.dot(xb, w1x_ref[...], preferred_element_type=jnp.float32)
         + jnp.dot(m, w1m_ref[...], preferred_element_type=jnp.float32)
         + b1_ref[...])
    h = jnp.where(h >= 0, h, 0.2 * h)
    h = jnp.dot(h, w2_ref[...], preferred_element_type=jnp.float32) + b2_ref[...]
    h = jnp.where(h >= 0, h, 0.2 * h)
    h = jnp.dot(h, w3_ref[...], preferred_element_type=jnp.float32)
    mu = jnp.mean(h, axis=-1, keepdims=True)
    var = jnp.mean((h - mu) ** 2, axis=-1, keepdims=True)
    o_ref[...] = (h - mu) * lax.rsqrt(var + 1e-5) * g_ref[...] + be_ref[...]


def _mlp_tc(x, parts, W1x, W1m, b1, W2, b2, W3, gamma, beta, block_n):
    n, d_in = x.shape
    d_msg = W1m.shape[0]
    d_out = W1x.shape[1]
    grid = (n // block_n,)
    return pl.pallas_call(
        _mlp_body,
        grid=grid,
        in_specs=[
            pl.BlockSpec((block_n, d_in), lambda i: (i, 0)),
            pl.BlockSpec((block_n, 128), lambda i: (i, 0)),
            pl.BlockSpec((d_in, d_out), lambda i: (0, 0)),
            pl.BlockSpec((d_msg, d_out), lambda i: (0, 0)),
            pl.BlockSpec((1, d_out), lambda i: (0, 0)),
            pl.BlockSpec((d_out, d_out), lambda i: (0, 0)),
            pl.BlockSpec((1, d_out), lambda i: (0, 0)),
            pl.BlockSpec((d_out, d_out), lambda i: (0, 0)),
            pl.BlockSpec((1, d_out), lambda i: (0, 0)),
            pl.BlockSpec((1, d_out), lambda i: (0, 0)),
        ],
        out_specs=pl.BlockSpec((block_n, d_out), lambda i: (i, 0)),
        out_shape=jax.ShapeDtypeStruct((n, d_out), jnp.float32),
        compiler_params=pltpu.CompilerParams(
            dimension_semantics=("arbitrary",),
        ),
    )(x, parts, W1x, W1m, b1, W2, b2, W3, gamma, beta)


def kernel(x, edge_index, message, W1, b1, W2, b2, W3, gamma, beta):
    n, d_in = x.shape
    e, d_msg = message.shape

    dest2d = edge_index[1].reshape(e // SUB, SUB)
    msg3 = (message.T.reshape(2, 8, e // SUB, SUB)
            .transpose(0, 2, 1, 3).reshape(-1, SUB))
    zeros = jnp.zeros((n, d_msg), dtype=jnp.float32)
    parts = _segment_sum_sc(dest2d, msg3, zeros)

    W1x = W1[:d_in]
    W1m = W1[d_in:]
    out = _mlp_tc(x, parts, W1x, W1m, b1.reshape(1, -1),
                  W2, b2.reshape(1, -1), W3, gamma.reshape(1, -1),
                  beta.reshape(1, -1), block_n=2000)
    return out
